# Initial kernel scaffold; baseline (speedup 1.0000x reference)
import jax
import jax.numpy as jnp
from jax.experimental import pallas as pl


def kernel(x, Wr, W1, b1, W2, b2, gamma, beta):
    raise NotImplementedError("write your pallas kernel here")


# speedup vs baseline: 1.1601x; 1.1601x over previous
import functools

import jax
import jax.numpy as jnp
from jax.experimental import pallas as pl
from jax.experimental.pallas import tpu as pltpu

D_MODEL = 1024
D_FF = 2048
NUM_EXPERTS = 8
TOP_K = 2

NEG = -1e30


def _router_body(x_ref, wr_ref, gates_ref):
    x = x_ref[...]
    logits = jnp.dot(x, wr_ref[...], preferred_element_type=jnp.float32)
    rows, lanes = logits.shape
    col = jax.lax.broadcasted_iota(jnp.int32, (rows, lanes), 1).astype(jnp.float32)
    valid = col < NUM_EXPERTS
    lm = jnp.where(valid, logits, NEG)
    m0 = jnp.max(lm, axis=-1, keepdims=True)
    idx0 = jnp.min(jnp.where(lm == m0, col, 1e9), axis=-1, keepdims=True)
    lm1 = jnp.where(col == idx0, NEG, lm)
    m1 = jnp.max(lm1, axis=-1, keepdims=True)
    idx1 = jnp.min(jnp.where(lm1 == m1, col, 1e9), axis=-1, keepdims=True)
    g0 = 1.0 / (1.0 + jnp.exp(m1 - m0))
    g1 = 1.0 / (1.0 + jnp.exp(m0 - m1))
    gates_ref[...] = jnp.where(col == idx0, g0, 0.0) + jnp.where(col == idx1, g1, 0.0)


def _ffn_body(x_ref, w1_ref, b1_ref, w2_ref, b2_ref, gamma_ref, beta_ref,
              gates_ref, out_ref, yacc_ref):
    e = pl.program_id(1)
    f = pl.program_id(2)
    nf = pl.num_programs(2)

    x = x_ref[...]
    h = jnp.dot(x, w1_ref[0], preferred_element_type=jnp.float32) + b1_ref[0, 0]
    h = h * 0.5 * (1.0 + jax.lax.erf(h * 0.7071067811865476))
    part = jnp.dot(h, w2_ref[0], preferred_element_type=jnp.float32)

    @pl.when(f == 0)
    def _():
        yacc_ref[...] = part

    @pl.when(f > 0)
    def _():
        yacc_ref[...] += part

    @pl.when(f == nf - 1)
    def _():
        t = x + yacc_ref[...] + b2_ref[0, 0]
        mu = jnp.mean(t, axis=-1, keepdims=True)
        d = t - mu
        var = jnp.mean(d * d, axis=-1, keepdims=True)
        z = d / jnp.sqrt(var + 1e-6) * gamma_ref[0, 0] + beta_ref[0, 0]
        col = jax.lax.broadcasted_iota(jnp.int32, gates_ref.shape, 1)
        gate = jnp.sum(jnp.where(col == e, gates_ref[...], 0.0), axis=-1,
                       keepdims=True)
        z = z * gate

        @pl.when(e == 0)
        def _():
            out_ref[...] = z

        @pl.when(e > 0)
        def _():
            out_ref[...] += z


def kernel(x, Wr, W1, b1, W2, b2, gamma, beta):
    B, S, D = x.shape
    x_flat = x.reshape(S, D)
    wr_pad = jnp.pad(Wr, ((0, 0), (0, 128 - NUM_EXPERTS)))

    RB = 512
    gates = pl.pallas_call(
        _router_body,
        grid=(S // RB,),
        in_specs=[
            pl.BlockSpec((RB, D), lambda r: (r, 0)),
            pl.BlockSpec((D, 128), lambda r: (0, 0)),
        ],
        out_specs=pl.BlockSpec((RB, 128), lambda r: (r, 0)),
        out_shape=jax.ShapeDtypeStruct((S, 128), jnp.float32),
    )(x_flat, wr_pad)

    RB2 = 256
    FB = 512
    out = pl.pallas_call(
        _ffn_body,
        grid=(S // RB2, NUM_EXPERTS, D_FF // FB),
        in_specs=[
            pl.BlockSpec((RB2, D), lambda r, e, f: (r, 0)),
            pl.BlockSpec((1, D, FB), lambda r, e, f: (e, 0, f)),
            pl.BlockSpec((1, 1, FB), lambda r, e, f: (e, 0, f)),
            pl.BlockSpec((1, FB, D), lambda r, e, f: (e, f, 0)),
            pl.BlockSpec((1, 1, D), lambda r, e, f: (e, 0, 0)),
            pl.BlockSpec((1, 1, D), lambda r, e, f: (e, 0, 0)),
            pl.BlockSpec((1, 1, D), lambda r, e, f: (e, 0, 0)),
            pl.BlockSpec((RB2, 128), lambda r, e, f: (r, 0)),
        ],
        out_specs=pl.BlockSpec((RB2, D), lambda r, e, f: (r, 0)),
        out_shape=jax.ShapeDtypeStruct((S, D), jnp.float32),
        scratch_shapes=[pltpu.VMEM((RB2, D), jnp.float32)],
    )(x_flat, W1, b1[:, None, :], W2, b2[:, None, :], gamma[:, None, :],
      beta[:, None, :], gates)

    return out.reshape(B, S, D)


# speedup vs baseline: 3.1332x; 2.7008x over previous
import functools

import jax
import jax.numpy as jnp
from jax import lax
from jax.experimental import pallas as pl
from jax.experimental.pallas import tpu as pltpu
from jax.experimental.pallas import tpu_sc as plsc

D_MODEL = 1024
D_FF = 2048
NUM_EXPERTS = 8
SEQ = 2048

RB = 256
NBLK = SEQ * 2 // RB + NUM_EXPERTS
NSLOT = NBLK * RB
PB = 256

NEG = -1e30


def _router_body(x_ref, wr_ref, pos0_ref, pos1_ref, g0_ref, g1_ref,
                 blk_ref, nblk_ref):
    x = x_ref[...]
    logits = jnp.dot(x, wr_ref[...], preferred_element_type=jnp.float32)
    rows, lanes = logits.shape
    col = lax.broadcasted_iota(jnp.int32, (rows, lanes), 1).astype(jnp.float32)
    valid = col < NUM_EXPERTS
    lm = jnp.where(valid, logits, NEG)
    m0 = jnp.max(lm, axis=-1, keepdims=True)
    idx0 = jnp.min(jnp.where(lm == m0, col, 1e9), axis=-1, keepdims=True)
    lm1 = jnp.where(col == idx0, NEG, lm)
    m1 = jnp.max(lm1, axis=-1, keepdims=True)
    idx1 = jnp.min(jnp.where(lm1 == m1, col, 1e9), axis=-1, keepdims=True)
    g0_ref[...] = 1.0 / (1.0 + jnp.exp(m1 - m0))
    g1_ref[...] = 1.0 / (1.0 + jnp.exp(m0 - m1))

    cnt = (jnp.where(col == idx0, 1.0, 0.0) + jnp.where(col == idx1, 1.0, 0.0))

    ri = lax.broadcasted_iota(jnp.int32, (PB, PB), 0)
    ci = lax.broadcasted_iota(jnp.int32, (PB, PB), 1)
    tstrict = jnp.where(ri > ci, 1.0, 0.0)
    carry = jnp.zeros((1, lanes), jnp.float32)
    prefixes = []
    for b in range(rows // PB):
        blk = lax.slice(cnt, (b * PB, 0), ((b + 1) * PB, lanes))
        prefixes.append(jnp.dot(tstrict, blk,
                                preferred_element_type=jnp.float32) + carry)
        carry = carry + jnp.sum(blk, axis=0, keepdims=True)

    totals = carry
    pc = jnp.ceil(totals * (1.0 / RB)) * RB
    ri2 = lax.broadcasted_iota(jnp.int32, (lanes, lanes), 0)
    ci2 = lax.broadcasted_iota(jnp.int32, (lanes, lanes), 1)
    lower = jnp.where(ri2 < ci2, 1.0, 0.0)
    off = jnp.dot(pc, lower, preferred_element_type=jnp.float32)
    off_end = off + pc

    for b in range(rows // PB):
        sl = (pl.ds(b * PB, PB), slice(None))
        colb = lax.slice(col, (b * PB, 0), ((b + 1) * PB, lanes))
        i0b = lax.slice(idx0, (b * PB, 0), ((b + 1) * PB, 1))
        i1b = lax.slice(idx1, (b * PB, 0), ((b + 1) * PB, 1))
        slot = off + prefixes[b]
        pos0_ref[sl] = jnp.sum(jnp.where(colb == i0b, slot, 0.0), axis=-1,
                               keepdims=True).astype(jnp.int32)
        pos1_ref[sl] = jnp.sum(jnp.where(colb == i1b, slot, 0.0), axis=-1,
                               keepdims=True).astype(jnp.int32)

    bp = (lax.broadcasted_iota(jnp.int32, (lanes, lanes), 0)
          .astype(jnp.float32) * RB)
    oe = jnp.broadcast_to(off_end, (lanes, lanes))
    colq = lax.broadcasted_iota(jnp.int32, (lanes, lanes), 1).astype(jnp.float32)
    done = jnp.where((oe <= bp) & (colq < NUM_EXPERTS), 1.0, 0.0)
    blk_ref[...] = jnp.minimum(
        jnp.sum(done, axis=-1, keepdims=True), NUM_EXPERTS - 1).astype(jnp.int32)
    nblk_ref[...] = (jnp.sum(pc, axis=-1, keepdims=True)
                     * (1.0 / RB)).astype(jnp.int32)


def _ffn_body(blk_s, nblk_s, xs_ref, w1_ref, b1_ref, w2_ref, b2_ref,
              gm_ref, bt_ref, sg_ref, ys_ref, w1bf_ref, w2bf_ref):
    b = pl.program_id(0)

    @pl.when(b < nblk_s[0])
    def _():
        prev = blk_s[jnp.maximum(b - 1, 0)]

        @pl.when((b == 0) | (blk_s[b] != prev))
        def _():
            w1bf_ref[...] = w1_ref[0].astype(jnp.bfloat16)
            w2bf_ref[...] = w2_ref[0].astype(jnp.bfloat16)

        xs = xs_ref[...]
        h = jnp.dot(xs.astype(jnp.bfloat16), w1bf_ref[...],
                    preferred_element_type=jnp.float32) + b1_ref[0, 0]
        h = h * 0.5 * (1.0 + lax.erf(h * 0.7071067811865476))
        part = jnp.dot(h.astype(jnp.bfloat16), w2bf_ref[...],
                       preferred_element_type=jnp.float32)
        t = xs + part + b2_ref[0, 0]
        mu = jnp.mean(t, axis=-1, keepdims=True)
        d = t - mu
        var = jnp.mean(d * d, axis=-1, keepdims=True)
        z = d / jnp.sqrt(var + 1e-6) * gm_ref[0, 0] + bt_ref[0, 0]
        ri = lax.broadcasted_iota(jnp.int32, (RB, RB), 0)
        ci = lax.broadcasted_iota(jnp.int32, (RB, RB), 1)
        g = jnp.broadcast_to(sg_ref[0], (RB, RB))
        gcol = jnp.sum(jnp.where(ri == ci, g, 0.0), axis=-1, keepdims=True)
        ys_ref[...] = z * gcol


def _make_sc_kernels():
    nc, ns = 2, 16
    nw = nc * ns
    spw = NSLOT // nw
    tpw = SEQ // nw
    tch = tpw // 2
    mesh = plsc.VectorSubcoreMesh(core_axis_name="c", subcore_axis_name="s",
                                  num_cores=nc, num_subcores=ns)

    @functools.partial(
        pl.kernel, mesh=mesh,
        compiler_params=pltpu.CompilerParams(needs_layout_passes=False),
        out_type=[
            jax.ShapeDtypeStruct((NSLOT, D_MODEL), jnp.float32),
            jax.ShapeDtypeStruct((NSLOT,), jnp.float32),
        ],
        scratch_types=[
            pltpu.VMEM((SEQ,), jnp.int32),
            pltpu.VMEM((SEQ,), jnp.int32),
            pltpu.VMEM((SEQ,), jnp.float32),
            pltpu.VMEM((SEQ,), jnp.float32),
            pltpu.VMEM((tpw,), jnp.int32),
            pltpu.VMEM((tpw,), jnp.int32),
            pltpu.VMEM((spw,), jnp.float32),
            pltpu.VMEM((tpw, D_MODEL), jnp.float32),
            pltpu.SemaphoreType.DMA,
        ],
    )
    def dispatch(x_hbm, pos0_hbm, pos1_hbm, g0_hbm, g1_hbm, xs_hbm, sg_hbm,
                 p0_v, p1_v, g0_v, g1_v, q0_v, q1_v, sg_v, xrow_v, sem):
        wid = lax.axis_index("s") * nc + lax.axis_index("c")
        tbase = wid * tpw
        base = wid * spw
        pltpu.sync_copy(x_hbm.at[pl.ds(tbase, tpw)], xrow_v)
        pltpu.sync_copy(pos0_hbm.at[pl.ds(tbase, tpw)], q0_v)
        pltpu.sync_copy(pos1_hbm.at[pl.ds(tbase, tpw)], q1_v)
        h0 = pltpu.async_copy(xrow_v, xs_hbm.at[q0_v], sem)
        h1 = pltpu.async_copy(xrow_v, xs_hbm.at[q1_v], sem)

        pltpu.sync_copy(pos0_hbm, p0_v)
        pltpu.sync_copy(pos1_hbm, p1_v)
        pltpu.sync_copy(g0_hbm, g0_v)
        pltpu.sync_copy(g1_hbm, g1_v)

        def scat(i, pv_ref, gv_ref):
            pv = pv_ref[pl.ds(i * 16, 16)]
            gv = gv_ref[pl.ds(i * 16, 16)]
            m = (pv >= base) & (pv < base + spw)
            idx = jnp.where(m, pv - base, 0)
            plsc.store_scatter(sg_v, [idx], gv, mask=m)
            return 0

        lax.fori_loop(0, SEQ // 16, lambda i, _: scat(i, p0_v, g0_v), 0)
        lax.fori_loop(0, SEQ // 16, lambda i, _: scat(i, p1_v, g1_v), 0)

        pltpu.sync_copy(sg_v, sg_hbm.at[pl.ds(base, spw)])
        h0.wait()
        h1.wait()

    @functools.partial(
        pl.kernel, mesh=mesh,
        out_type=jax.ShapeDtypeStruct((SEQ, D_MODEL), jnp.float32),
        scratch_types=[
            pltpu.VMEM((tch,), jnp.int32),
            pltpu.VMEM((tch,), jnp.int32),
            pltpu.VMEM((tch, D_MODEL), jnp.float32),
            pltpu.VMEM((tch, D_MODEL), jnp.float32),
            pltpu.SemaphoreType.DMA,
        ],
    )
    def combine(ys_hbm, pos0_hbm, pos1_hbm, out_hbm, p0_v, p1_v, buf0, buf1,
                sem):
        wid = lax.axis_index("s") * nc + lax.axis_index("c")
        for c in range(tpw // tch):
            tbase = wid * tpw + c * tch
            pltpu.sync_copy(pos0_hbm.at[pl.ds(tbase, tch)], p0_v)
            pltpu.sync_copy(pos1_hbm.at[pl.ds(tbase, tch)], p1_v)
            h0 = pltpu.async_copy(ys_hbm.at[p0_v], buf0, sem)
            h1 = pltpu.async_copy(ys_hbm.at[p1_v], buf1, sem)
            h0.wait()
            h1.wait()

            def row(i, _):
                for j in range(D_MODEL // 16):
                    sl = (i, pl.ds(j * 16, 16))
                    buf0[sl] = buf0[sl] + buf1[sl]
                return 0

            lax.fori_loop(0, tch, row, 0)
            pltpu.sync_copy(buf0, out_hbm.at[pl.ds(tbase, tch)])

    return dispatch, combine


_SC_KERNELS = []


def _sc_kernels():
    if not _SC_KERNELS:
        _SC_KERNELS.extend(_make_sc_kernels())
    return _SC_KERNELS


def kernel(x, Wr, W1, b1, W2, b2, gamma, beta):
    B, S, D = x.shape
    x_flat = x.reshape(S, D)
    wr_pad = jnp.pad(Wr, ((0, 0), (0, 128 - NUM_EXPERTS)))

    pos0, pos1, g0, g1, blk, nblk = pl.pallas_call(
        _router_body,
        out_shape=[
            jax.ShapeDtypeStruct((S, 1), jnp.int32),
            jax.ShapeDtypeStruct((S, 1), jnp.int32),
            jax.ShapeDtypeStruct((S, 1), jnp.float32),
            jax.ShapeDtypeStruct((S, 1), jnp.float32),
            jax.ShapeDtypeStruct((128, 1), jnp.int32),
            jax.ShapeDtypeStruct((1, 1), jnp.int32),
        ],
    )(x_flat, wr_pad)

    pos0 = pos0.reshape(S)
    pos1 = pos1.reshape(S)
    g0 = g0.reshape(S)
    g1 = g1.reshape(S)

    dispatch, combine = _sc_kernels()
    xs, sg = dispatch(x_flat, pos0, pos1, g0, g1)

    ys = pl.pallas_call(
        _ffn_body,
        grid_spec=pltpu.PrefetchScalarGridSpec(
            num_scalar_prefetch=2,
            grid=(NBLK,),
            in_specs=[
                pl.BlockSpec((RB, D), lambda b, blk, nb: (b, 0)),
                pl.BlockSpec((1, D, D_FF), lambda b, blk, nb: (blk[b], 0, 0)),
                pl.BlockSpec((1, 1, D_FF), lambda b, blk, nb: (blk[b], 0, 0)),
                pl.BlockSpec((1, D_FF, D), lambda b, blk, nb: (blk[b], 0, 0)),
                pl.BlockSpec((1, 1, D), lambda b, blk, nb: (blk[b], 0, 0)),
                pl.BlockSpec((1, 1, D), lambda b, blk, nb: (blk[b], 0, 0)),
                pl.BlockSpec((1, 1, D), lambda b, blk, nb: (blk[b], 0, 0)),
                pl.BlockSpec((1, 1, RB), lambda b, blk, nb: (b, 0, 0)),
            ],
            out_specs=pl.BlockSpec((RB, D), lambda b, blk, nb: (b, 0)),
            scratch_shapes=[pltpu.VMEM((D, D_FF), jnp.bfloat16),
                            pltpu.VMEM((D_FF, D), jnp.bfloat16)],
        ),
        out_shape=jax.ShapeDtypeStruct((NSLOT, D), jnp.float32),
    )(blk.reshape(128), nblk.reshape(1), xs, W1, b1[:, None, :], W2,
      b2[:, None, :], gamma[:, None, :], beta[:, None, :],
      sg.reshape(NBLK, 1, RB))

    out = combine(ys, pos0, pos1)
    return out.reshape(B, S, D)


# speedup vs baseline: 3.2656x; 1.0423x over previous
import functools

import jax
import jax.numpy as jnp
from jax import lax
from jax.experimental import pallas as pl
from jax.experimental.pallas import tpu as pltpu
from jax.experimental.pallas import tpu_sc as plsc

D_MODEL = 1024
D_FF = 2048
NUM_EXPERTS = 8
SEQ = 2048

RB = 256
NBLK = SEQ * 2 // RB + NUM_EXPERTS
NSLOT = NBLK * RB
PB = 256

NEG = -1e30


def _router_body(x_ref, wr_ref, pos0_ref, pos1_ref, g0_ref, g1_ref,
                 blk_ref, nblk_ref):
    x = x_ref[...]
    logits = jnp.dot(x, wr_ref[...], preferred_element_type=jnp.float32)
    rows, lanes = logits.shape
    col = lax.broadcasted_iota(jnp.int32, (rows, lanes), 1).astype(jnp.float32)
    valid = col < NUM_EXPERTS
    lm = jnp.where(valid, logits, NEG)
    m0 = jnp.max(lm, axis=-1, keepdims=True)
    idx0 = jnp.min(jnp.where(lm == m0, col, 1e9), axis=-1, keepdims=True)
    lm1 = jnp.where(col == idx0, NEG, lm)
    m1 = jnp.max(lm1, axis=-1, keepdims=True)
    idx1 = jnp.min(jnp.where(lm1 == m1, col, 1e9), axis=-1, keepdims=True)
    g0_ref[...] = 1.0 / (1.0 + jnp.exp(m1 - m0))
    g1_ref[...] = 1.0 / (1.0 + jnp.exp(m0 - m1))

    cnt = (jnp.where(col == idx0, 1.0, 0.0) + jnp.where(col == idx1, 1.0, 0.0))

    ri = lax.broadcasted_iota(jnp.int32, (PB, PB), 0)
    ci = lax.broadcasted_iota(jnp.int32, (PB, PB), 1)
    tstrict = jnp.where(ri > ci, 1.0, 0.0)
    carry = jnp.zeros((1, lanes), jnp.float32)
    prefixes = []
    for b in range(rows // PB):
        blk = lax.slice(cnt, (b * PB, 0), ((b + 1) * PB, lanes))
        prefixes.append(jnp.dot(tstrict, blk,
                                preferred_element_type=jnp.float32) + carry)
        carry = carry + jnp.sum(blk, axis=0, keepdims=True)

    totals = carry
    pc = jnp.ceil(totals * (1.0 / RB)) * RB
    ri2 = lax.broadcasted_iota(jnp.int32, (lanes, lanes), 0)
    ci2 = lax.broadcasted_iota(jnp.int32, (lanes, lanes), 1)
    lower = jnp.where(ri2 < ci2, 1.0, 0.0)
    off = jnp.dot(pc, lower, preferred_element_type=jnp.float32)
    off_end = off + pc

    for b in range(rows // PB):
        sl = (pl.ds(b * PB, PB), slice(None))
        colb = lax.slice(col, (b * PB, 0), ((b + 1) * PB, lanes))
        i0b = lax.slice(idx0, (b * PB, 0), ((b + 1) * PB, 1))
        i1b = lax.slice(idx1, (b * PB, 0), ((b + 1) * PB, 1))
        slot = off + prefixes[b]
        pos0_ref[sl] = jnp.sum(jnp.where(colb == i0b, slot, 0.0), axis=-1,
                               keepdims=True).astype(jnp.int32)
        pos1_ref[sl] = jnp.sum(jnp.where(colb == i1b, slot, 0.0), axis=-1,
                               keepdims=True).astype(jnp.int32)

    bp = (lax.broadcasted_iota(jnp.int32, (lanes, lanes), 0)
          .astype(jnp.float32) * RB)
    oe = jnp.broadcast_to(off_end, (lanes, lanes))
    colq = lax.broadcasted_iota(jnp.int32, (lanes, lanes), 1).astype(jnp.float32)
    done = jnp.where((oe <= bp) & (colq < NUM_EXPERTS), 1.0, 0.0)
    blk_ref[...] = jnp.minimum(
        jnp.sum(done, axis=-1, keepdims=True), NUM_EXPERTS - 1).astype(jnp.int32)
    nblk_ref[...] = (jnp.sum(pc, axis=-1, keepdims=True)
                     * (1.0 / RB)).astype(jnp.int32)


def _ffn_body(blk_s, nblk_s, xs_ref, w1_ref, b1_ref, w2_ref, b2_ref,
              gm_ref, bt_ref, sg_ref, ys_ref):
    b = pl.program_id(0)

    @pl.when(b < nblk_s[0])
    def _():
        xs = xs_ref[...]
        h = jnp.dot(xs, w1_ref[0],
                    precision=lax.Precision.DEFAULT,
                    preferred_element_type=jnp.float32) + b1_ref[0, 0]
        h = h * 0.5 * (1.0 + lax.erf(h * 0.7071067811865476))
        part = jnp.dot(h, w2_ref[0],
                       precision=lax.Precision.DEFAULT,
                       preferred_element_type=jnp.float32)
        t = xs + part + b2_ref[0, 0]
        mu = jnp.mean(t, axis=-1, keepdims=True)
        d = t - mu
        var = jnp.mean(d * d, axis=-1, keepdims=True)
        z = d / jnp.sqrt(var + 1e-6) * gm_ref[0, 0] + bt_ref[0, 0]
        ri = lax.broadcasted_iota(jnp.int32, (RB, RB), 0)
        ci = lax.broadcasted_iota(jnp.int32, (RB, RB), 1)
        g = jnp.broadcast_to(sg_ref[0], (RB, RB))
        gcol = jnp.sum(jnp.where(ri == ci, g, 0.0), axis=-1, keepdims=True)
        ys_ref[...] = z * gcol


def _make_sc_kernels():
    nc, ns = 2, 16
    nw = nc * ns
    spw = NSLOT // nw
    tpw = SEQ // nw
    tch = tpw // 2
    mesh = plsc.VectorSubcoreMesh(core_axis_name="c", subcore_axis_name="s",
                                  num_cores=nc, num_subcores=ns)

    @functools.partial(
        pl.kernel, mesh=mesh,
        compiler_params=pltpu.CompilerParams(needs_layout_passes=False),
        out_type=[
            jax.ShapeDtypeStruct((NSLOT, D_MODEL), jnp.float32),
            jax.ShapeDtypeStruct((NSLOT,), jnp.float32),
        ],
        scratch_types=[
            pltpu.VMEM((SEQ,), jnp.int32),
            pltpu.VMEM((SEQ,), jnp.int32),
            pltpu.VMEM((SEQ,), jnp.float32),
            pltpu.VMEM((SEQ,), jnp.float32),
            pltpu.VMEM((tpw,), jnp.int32),
            pltpu.VMEM((tpw,), jnp.int32),
            pltpu.VMEM((spw,), jnp.float32),
            pltpu.VMEM((tpw, D_MODEL), jnp.float32),
            pltpu.SemaphoreType.DMA,
        ],
    )
    def dispatch(x_hbm, pos0_hbm, pos1_hbm, g0_hbm, g1_hbm, xs_hbm, sg_hbm,
                 p0_v, p1_v, g0_v, g1_v, q0_v, q1_v, sg_v, xrow_v, sem):
        wid = lax.axis_index("s") * nc + lax.axis_index("c")
        tbase = wid * tpw
        base = wid * spw
        pltpu.sync_copy(x_hbm.at[pl.ds(tbase, tpw)], xrow_v)
        pltpu.sync_copy(pos0_hbm.at[pl.ds(tbase, tpw)], q0_v)
        pltpu.sync_copy(pos1_hbm.at[pl.ds(tbase, tpw)], q1_v)
        h0 = pltpu.async_copy(xrow_v, xs_hbm.at[q0_v], sem)
        h1 = pltpu.async_copy(xrow_v, xs_hbm.at[q1_v], sem)

        pltpu.sync_copy(pos0_hbm, p0_v)
        pltpu.sync_copy(pos1_hbm, p1_v)
        pltpu.sync_copy(g0_hbm, g0_v)
        pltpu.sync_copy(g1_hbm, g1_v)

        def scat(i, pv_ref, gv_ref):
            pv = pv_ref[pl.ds(i * 16, 16)]
            gv = gv_ref[pl.ds(i * 16, 16)]
            m = (pv >= base) & (pv < base + spw)
            idx = jnp.where(m, pv - base, 0)
            plsc.store_scatter(sg_v, [idx], gv, mask=m)
            return 0

        lax.fori_loop(0, SEQ // 16, lambda i, _: scat(i, p0_v, g0_v), 0)
        lax.fori_loop(0, SEQ // 16, lambda i, _: scat(i, p1_v, g1_v), 0)

        pltpu.sync_copy(sg_v, sg_hbm.at[pl.ds(base, spw)])
        h0.wait()
        h1.wait()

    @functools.partial(
        pl.kernel, mesh=mesh,
        out_type=jax.ShapeDtypeStruct((SEQ, D_MODEL), jnp.float32),
        scratch_types=[
            pltpu.VMEM((tch,), jnp.int32),
            pltpu.VMEM((tch,), jnp.int32),
            pltpu.VMEM((tch, D_MODEL), jnp.float32),
            pltpu.VMEM((tch, D_MODEL), jnp.float32),
            pltpu.SemaphoreType.DMA,
        ],
    )
    def combine(ys_hbm, pos0_hbm, pos1_hbm, out_hbm, p0_v, p1_v, buf0, buf1,
                sem):
        wid = lax.axis_index("s") * nc + lax.axis_index("c")
        for c in range(tpw // tch):
            tbase = wid * tpw + c * tch
            pltpu.sync_copy(pos0_hbm.at[pl.ds(tbase, tch)], p0_v)
            pltpu.sync_copy(pos1_hbm.at[pl.ds(tbase, tch)], p1_v)
            h0 = pltpu.async_copy(ys_hbm.at[p0_v], buf0, sem)
            h1 = pltpu.async_copy(ys_hbm.at[p1_v], buf1, sem)
            h0.wait()
            h1.wait()

            def row(i, _):
                for j in range(D_MODEL // 16):
                    sl = (i, pl.ds(j * 16, 16))
                    buf0[sl] = buf0[sl] + buf1[sl]
                return 0

            lax.fori_loop(0, tch, row, 0)
            pltpu.sync_copy(buf0, out_hbm.at[pl.ds(tbase, tch)])

    return dispatch, combine


_SC_KERNELS = []


def _sc_kernels():
    if not _SC_KERNELS:
        _SC_KERNELS.extend(_make_sc_kernels())
    return _SC_KERNELS


def kernel(x, Wr, W1, b1, W2, b2, gamma, beta):
    B, S, D = x.shape
    x_flat = x.reshape(S, D)
    wr_pad = jnp.pad(Wr, ((0, 0), (0, 128 - NUM_EXPERTS)))

    pos0, pos1, g0, g1, blk, nblk = pl.pallas_call(
        _router_body,
        out_shape=[
            jax.ShapeDtypeStruct((S, 1), jnp.int32),
            jax.ShapeDtypeStruct((S, 1), jnp.int32),
            jax.ShapeDtypeStruct((S, 1), jnp.float32),
            jax.ShapeDtypeStruct((S, 1), jnp.float32),
            jax.ShapeDtypeStruct((128, 1), jnp.int32),
            jax.ShapeDtypeStruct((1, 1), jnp.int32),
        ],
    )(x_flat, wr_pad)

    pos0 = pos0.reshape(S)
    pos1 = pos1.reshape(S)
    g0 = g0.reshape(S)
    g1 = g1.reshape(S)

    dispatch, combine = _sc_kernels()
    xs, sg = dispatch(x_flat, pos0, pos1, g0, g1)

    ys = pl.pallas_call(
        _ffn_body,
        grid_spec=pltpu.PrefetchScalarGridSpec(
            num_scalar_prefetch=2,
            grid=(NBLK,),
            in_specs=[
                pl.BlockSpec((RB, D), lambda b, blk, nb: (b, 0)),
                pl.BlockSpec((1, D, D_FF), lambda b, blk, nb: (blk[b], 0, 0)),
                pl.BlockSpec((1, 1, D_FF), lambda b, blk, nb: (blk[b], 0, 0)),
                pl.BlockSpec((1, D_FF, D), lambda b, blk, nb: (blk[b], 0, 0)),
                pl.BlockSpec((1, 1, D), lambda b, blk, nb: (blk[b], 0, 0)),
                pl.BlockSpec((1, 1, D), lambda b, blk, nb: (blk[b], 0, 0)),
                pl.BlockSpec((1, 1, D), lambda b, blk, nb: (blk[b], 0, 0)),
                pl.BlockSpec((1, 1, RB), lambda b, blk, nb: (b, 0, 0)),
            ],
            out_specs=pl.BlockSpec((RB, D), lambda b, blk, nb: (b, 0)),
        ),
        out_shape=jax.ShapeDtypeStruct((NSLOT, D), jnp.float32),
    )(blk.reshape(128), nblk.reshape(1), xs, W1, b1[:, None, :], W2,
      b2[:, None, :], gamma[:, None, :], beta[:, None, :],
      sg.reshape(NBLK, 1, RB))

    out = combine(ys, pos0, pos1)
    return out.reshape(B, S, D)
